# Initial kernel scaffold; baseline (speedup 1.0000x reference)
import jax
import jax.numpy as jnp
from jax.experimental import pallas as pl


def kernel(entity_ids, entity_table):
    raise NotImplementedError("write your pallas kernel here")


# speedup vs baseline: 1.9774x; 1.9774x over previous
import functools

import jax
import jax.numpy as jnp
from jax import lax
from jax.experimental import pallas as pl
from jax.experimental.pallas import tpu as pltpu
from jax.experimental.pallas import tpu_sc as plsc

NUM_CORES = 2
NUM_SUBCORES = 16
LANES = 16
NUM_WORKERS = NUM_CORES * NUM_SUBCORES


def kernel(entity_ids, entity_table):
    (batch,) = entity_ids.shape
    vocab, dim = entity_table.shape
    per_worker = batch // NUM_WORKERS

    ids32 = entity_ids.astype(jnp.int32)
    table_flat = entity_table.reshape(vocab * dim)

    mesh = plsc.VectorSubcoreMesh(
        core_axis_name="c",
        subcore_axis_name="s",
        num_cores=NUM_CORES,
        num_subcores=NUM_SUBCORES,
    )

    @functools.partial(
        pl.kernel,
        out_type=jax.ShapeDtypeStruct((batch * dim,), jnp.float32),
        mesh=mesh,
        compiler_params=pltpu.CompilerParams(needs_layout_passes=False),
        scratch_types=[
            pltpu.VMEM((per_worker,), jnp.int32),
            pltpu.VMEM((vocab * dim,), jnp.float32),
            pltpu.VMEM((per_worker * dim,), jnp.float32),
        ],
    )
    def gather_kernel(ids_hbm, table_hbm, out_hbm, ids_v, table_v, out_v):
        wid = lax.axis_index("s") * NUM_CORES + lax.axis_index("c")
        base = wid * per_worker
        pltpu.sync_copy(ids_hbm.at[pl.ds(base, per_worker)], ids_v)
        pltpu.sync_copy(table_hbm, table_v)
        lane = lax.broadcasted_iota(jnp.int32, (LANES,), 0)
        for i in range(per_worker // LANES):
            flat = ids_v[pl.ds(i * LANES, LANES)] * dim
            pos = (i * LANES) * dim + lane * dim
            for d in range(dim):
                vals = plsc.load_gather(table_v, [flat + d])
                plsc.store_scatter(out_v, [pos + d], vals)
        pltpu.sync_copy(out_v, out_hbm.at[pl.ds(base * dim, per_worker * dim)])

    return gather_kernel(ids32, table_flat).reshape(batch, dim)


# speedup vs baseline: 2.0094x; 1.0162x over previous
import functools

import jax
import jax.numpy as jnp
from jax import lax
from jax.experimental import pallas as pl
from jax.experimental.pallas import tpu as pltpu
from jax.experimental.pallas import tpu_sc as plsc

NUM_CORES = 2
NUM_SUBCORES = 16
LANES = 16
NUM_WORKERS = NUM_CORES * NUM_SUBCORES


def kernel(entity_ids, entity_table):
    (batch,) = entity_ids.shape
    vocab, dim = entity_table.shape
    per_worker = batch // NUM_WORKERS

    ids32 = entity_ids.astype(jnp.int32)
    table_flat = entity_table.reshape(vocab * dim)

    mesh = plsc.VectorSubcoreMesh(
        core_axis_name="c",
        subcore_axis_name="s",
        num_cores=NUM_CORES,
        num_subcores=NUM_SUBCORES,
    )

    @functools.partial(
        pl.kernel,
        out_type=jax.ShapeDtypeStruct((batch * dim,), jnp.float32),
        mesh=mesh,
        compiler_params=pltpu.CompilerParams(needs_layout_passes=False),
        scratch_types=[
            pltpu.VMEM((per_worker,), jnp.int32),
            pltpu.VMEM((vocab * dim,), jnp.float32),
            pltpu.VMEM((per_worker * dim,), jnp.float32),
        ],
    )
    def gather_kernel(ids_hbm, table_hbm, out_hbm, ids_v, table_v, out_v):
        wid = lax.axis_index("s") * NUM_CORES + lax.axis_index("c")
        base = wid * per_worker
        pltpu.sync_copy(ids_hbm.at[pl.ds(base, per_worker)], ids_v)
        pltpu.sync_copy(table_hbm, table_v)
        lane = lax.broadcasted_iota(jnp.int32, (LANES,), 0)

        def body(i, _):
            flat = ids_v[pl.ds(i * LANES, LANES)] * dim
            pos = i * (LANES * dim) + lane * dim
            for d in range(dim):
                vals = plsc.load_gather(table_v, [flat + d])
                plsc.store_scatter(out_v, [pos + d], vals)
            return 0

        lax.fori_loop(0, per_worker // LANES, body, 0, unroll=4)
        pltpu.sync_copy(out_v, out_hbm.at[pl.ds(base * dim, per_worker * dim)])

    return gather_kernel(ids32, table_flat).reshape(batch, dim)


# speedup vs baseline: 2.0604x; 1.0254x over previous
import functools

import jax
import jax.numpy as jnp
from jax import lax
from jax.experimental import pallas as pl
from jax.experimental.pallas import tpu as pltpu
from jax.experimental.pallas import tpu_sc as plsc

NUM_CORES = 1
NUM_SUBCORES = 16
LANES = 16
NUM_WORKERS = NUM_CORES * NUM_SUBCORES


def kernel(entity_ids, entity_table):
    (batch,) = entity_ids.shape
    vocab, dim = entity_table.shape
    per_worker = batch // NUM_WORKERS

    ids32 = entity_ids.astype(jnp.int32)
    table_flat = entity_table.reshape(vocab * dim)

    mesh = plsc.VectorSubcoreMesh(
        core_axis_name="c",
        subcore_axis_name="s",
        num_cores=NUM_CORES,
        num_subcores=NUM_SUBCORES,
    )

    @functools.partial(
        pl.kernel,
        out_type=jax.ShapeDtypeStruct((batch * dim,), jnp.float32),
        mesh=mesh,
        compiler_params=pltpu.CompilerParams(needs_layout_passes=False),
        scratch_types=[
            pltpu.VMEM((per_worker,), jnp.int32),
            pltpu.VMEM((vocab * dim,), jnp.float32),
            pltpu.VMEM((per_worker * dim,), jnp.float32),
        ],
    )
    def gather_kernel(ids_hbm, table_hbm, out_hbm, ids_v, table_v, out_v):
        wid = lax.axis_index("s") * NUM_CORES + lax.axis_index("c")
        base = wid * per_worker
        pltpu.sync_copy(ids_hbm.at[pl.ds(base, per_worker)], ids_v)
        pltpu.sync_copy(table_hbm, table_v)
        lane = lax.broadcasted_iota(jnp.int32, (LANES,), 0)

        def body(i, _):
            flat = ids_v[pl.ds(i * LANES, LANES)] * dim
            pos = i * (LANES * dim) + lane * dim
            for d in range(dim):
                vals = plsc.load_gather(table_v, [flat + d])
                plsc.store_scatter(out_v, [pos + d], vals)
            return 0

        lax.fori_loop(0, per_worker // LANES, body, 0, unroll=4)
        pltpu.sync_copy(out_v, out_hbm.at[pl.ds(base * dim, per_worker * dim)])

    return gather_kernel(ids32, table_flat).reshape(batch, dim)


# speedup vs baseline: 3.0479x; 1.4793x over previous
import functools

import jax
import jax.numpy as jnp
from jax import lax
from jax.experimental import pallas as pl
from jax.experimental.pallas import tpu as pltpu
from jax.experimental.pallas import tpu_sc as plsc

NUM_CORES = 1
NUM_SUBCORES = 16
LANES = 16
NUM_WORKERS = NUM_CORES * NUM_SUBCORES


def kernel(entity_ids, entity_table):
    (batch,) = entity_ids.shape
    vocab, dim = entity_table.shape
    per_worker = batch // NUM_WORKERS

    ids32 = entity_ids.astype(jnp.int32)

    mesh = plsc.VectorSubcoreMesh(
        core_axis_name="c",
        subcore_axis_name="s",
        num_cores=NUM_CORES,
        num_subcores=NUM_SUBCORES,
    )

    @functools.partial(
        pl.kernel,
        out_type=tuple(
            jax.ShapeDtypeStruct((batch,), jnp.float32) for _ in range(dim)
        ),
        mesh=mesh,
        compiler_params=pltpu.CompilerParams(needs_layout_passes=False),
        scratch_types=[
            pltpu.VMEM((per_worker,), jnp.int32),
            pltpu.VMEM((vocab, dim), jnp.float32),
        ]
        + [pltpu.VMEM((per_worker,), jnp.float32) for _ in range(dim)],
    )
    def gather_kernel(ids_hbm, table_hbm, c0_hbm, c1_hbm, c2_hbm,
                      ids_v, table_v, c0_v, c1_v, c2_v):
        cols_hbm = (c0_hbm, c1_hbm, c2_hbm)
        cols_v = (c0_v, c1_v, c2_v)
        wid = lax.axis_index("s") * NUM_CORES + lax.axis_index("c")
        base = wid * per_worker
        pltpu.sync_copy(ids_hbm.at[pl.ds(base, per_worker)], ids_v)
        pltpu.sync_copy(table_hbm, table_v)

        def body(i, _):
            rows = ids_v[pl.ds(i * LANES, LANES)]
            for d in range(dim):
                col = jnp.full((LANES,), d, jnp.int32)
                cols_v[d][pl.ds(i * LANES, LANES)] = plsc.load_gather(
                    table_v, [rows, col]
                )
            return 0

        lax.fori_loop(0, per_worker // LANES, body, 0, unroll=4)
        for d in range(dim):
            pltpu.sync_copy(cols_v[d], cols_hbm[d].at[pl.ds(base, per_worker)])

    cols = gather_kernel(ids32, entity_table)
    return jnp.stack(cols, axis=1)
